# Initial kernel scaffold; baseline (speedup 1.0000x reference)
import jax
import jax.numpy as jnp
from jax.experimental import pallas as pl


def kernel(x, gate_w, dense_1_w, dense_1_b, dense_2_w, dense_2_b):
    raise NotImplementedError("write your pallas kernel here")


# speedup vs baseline: 2.1488x; 2.1488x over previous
import functools

import jax
import jax.numpy as jnp
from jax.experimental import pallas as pl
from jax.experimental.pallas import tpu as pltpu

_N_EXPERTS = 8
_FFW = 1024
_D = 768
_LIMIT = 7.0
_ALPHA = 1.702


def _moe_body(x_ref, gw_ref, w1g_ref, w1x_ref, b1g_ref, b1x_ref, w2_ref,
              b2_ref, out_ref):
    e = pl.program_id(0)
    x = x_ref[...]

    gate = jnp.dot(x, gw_ref[...], preferred_element_type=jnp.float32)
    cols = jax.lax.broadcasted_iota(jnp.int32, gate.shape, 1)
    m1 = jnp.max(gate, axis=1, keepdims=True)
    a1 = jnp.min(jnp.where(gate == m1, cols, _N_EXPERTS), axis=1, keepdims=True)
    gate2 = jnp.where(cols == a1, -jnp.inf, gate)
    m2 = jnp.max(gate2, axis=1, keepdims=True)
    a2 = jnp.min(jnp.where(gate2 == m2, cols, _N_EXPERTS), axis=1, keepdims=True)
    e2 = jnp.exp(m2 - m1)
    denom = 1.0 + e2
    w_sel = jnp.where(a1 == e, 1.0 / denom,
                      jnp.where(a2 == e, e2 / denom, 0.0))

    dn = (((1,), (1,)), ((), ()))
    hg = jax.lax.dot_general(x, w1g_ref[0], dn,
                             preferred_element_type=jnp.float32) + b1g_ref[0]
    hx = jax.lax.dot_general(x, w1x_ref[0], dn,
                             preferred_element_type=jnp.float32) + b1x_ref[0]
    g = jnp.minimum(hg, _LIMIT)
    xl = jnp.clip(hx, -_LIMIT, _LIMIT)
    act = g * jax.nn.sigmoid(_ALPHA * g) * (xl + 1.0)
    h2 = jax.lax.dot_general(act, w2_ref[0], dn,
                             preferred_element_type=jnp.float32) + b2_ref[0]
    contrib = w_sel * h2

    @pl.when(e == 0)
    def _():
        out_ref[...] = contrib

    @pl.when(e != 0)
    def _():
        out_ref[...] += contrib


@jax.jit
def kernel(x, gate_w, dense_1_w, dense_1_b, dense_2_w, dense_2_b):
    B, L, D = x.shape
    T = B * L
    x_f = x.reshape(T, D)
    w1g = dense_1_w[:, 0::2, :]
    w1x = dense_1_w[:, 1::2, :]
    b1g = dense_1_b[:, 0::2].reshape(_N_EXPERTS, 1, _FFW)
    b1x = dense_1_b[:, 1::2].reshape(_N_EXPERTS, 1, _FFW)
    b2 = dense_2_b.reshape(_N_EXPERTS, 1, _D)

    out = pl.pallas_call(
        _moe_body,
        grid=(_N_EXPERTS,),
        in_specs=[
            pl.BlockSpec((T, D), lambda e: (0, 0)),
            pl.BlockSpec((D, _N_EXPERTS), lambda e: (0, 0)),
            pl.BlockSpec((1, _FFW, D), lambda e: (e, 0, 0)),
            pl.BlockSpec((1, _FFW, D), lambda e: (e, 0, 0)),
            pl.BlockSpec((1, 1, _FFW), lambda e: (e, 0, 0)),
            pl.BlockSpec((1, 1, _FFW), lambda e: (e, 0, 0)),
            pl.BlockSpec((1, D, _FFW), lambda e: (e, 0, 0)),
            pl.BlockSpec((1, 1, D), lambda e: (e, 0, 0)),
        ],
        out_specs=pl.BlockSpec((T, D), lambda e: (0, 0)),
        out_shape=jax.ShapeDtypeStruct((T, D), jnp.float32),
        compiler_params=pltpu.CompilerParams(
            dimension_semantics=("arbitrary",)),
    )(x_f, gate_w, w1g, w1x, b1g, b1x, dense_2_w, b2)
    return out.reshape(B, L, D)


# speedup vs baseline: 6.5385x; 3.0429x over previous
import functools

import jax
import jax.numpy as jnp
from jax.experimental import pallas as pl
from jax.experimental.pallas import tpu as pltpu

_N_EXPERTS = 8
_FFW = 1024
_D = 768
_LIMIT = 7.0
_ALPHA = 1.702


def _moe_body(x_ref, gw_ref, w1_ref, b1g_ref, b1x_ref, w2_ref,
              b2_ref, out_ref):
    e = pl.program_id(0)
    x = x_ref[...]

    gate = jnp.dot(x, gw_ref[...], preferred_element_type=jnp.float32)
    cols = jax.lax.broadcasted_iota(jnp.int32, gate.shape, 1)
    m1 = jnp.max(gate, axis=1, keepdims=True)
    a1 = jnp.min(jnp.where(gate == m1, cols, _N_EXPERTS), axis=1, keepdims=True)
    gate2 = jnp.where(cols == a1, -jnp.inf, gate)
    m2 = jnp.max(gate2, axis=1, keepdims=True)
    a2 = jnp.min(jnp.where(gate2 == m2, cols, _N_EXPERTS), axis=1, keepdims=True)
    e2 = jnp.exp(m2 - m1)
    denom = 1.0 + e2
    w_sel = jnp.where(a1 == e, 1.0 / denom,
                      jnp.where(a2 == e, e2 / denom, 0.0))

    dn = (((1,), (1,)), ((), ()))
    w1g = w1_ref[0, :, 0:_D]
    w1x = w1_ref[0, :, _D:2 * _D]
    hg = jax.lax.dot_general(x, w1g, dn,
                             preferred_element_type=jnp.float32) + b1g_ref[0]
    hx = jax.lax.dot_general(x, w1x, dn,
                             preferred_element_type=jnp.float32) + b1x_ref[0]
    g = jnp.minimum(hg, _LIMIT)
    xl = jnp.clip(hx, -_LIMIT, _LIMIT)
    act = g * jax.nn.sigmoid(_ALPHA * g) * (xl + 1.0)
    h2 = jax.lax.dot_general(act, w2_ref[0], dn,
                             preferred_element_type=jnp.float32) + b2_ref[0]
    contrib = w_sel * h2

    @pl.when(e == 0)
    def _():
        out_ref[...] = contrib

    @pl.when(e != 0)
    def _():
        out_ref[...] += contrib


@jax.jit
def kernel(x, gate_w, dense_1_w, dense_1_b, dense_2_w, dense_2_b):
    B, L, D = x.shape
    T = B * L
    x_f = x.reshape(T, D)
    w1r = dense_1_w.reshape(_N_EXPERTS, _FFW, 2 * D)
    b1g = dense_1_b[:, 0::2].reshape(_N_EXPERTS, 1, _FFW)
    b1x = dense_1_b[:, 1::2].reshape(_N_EXPERTS, 1, _FFW)
    b2 = dense_2_b.reshape(_N_EXPERTS, 1, _D)

    out = pl.pallas_call(
        _moe_body,
        grid=(_N_EXPERTS,),
        in_specs=[
            pl.BlockSpec((T, D), lambda e: (0, 0)),
            pl.BlockSpec((D, _N_EXPERTS), lambda e: (0, 0)),
            pl.BlockSpec((1, _FFW, 2 * D), lambda e: (e, 0, 0)),
            pl.BlockSpec((1, 1, _FFW), lambda e: (e, 0, 0)),
            pl.BlockSpec((1, 1, _FFW), lambda e: (e, 0, 0)),
            pl.BlockSpec((1, D, _FFW), lambda e: (e, 0, 0)),
            pl.BlockSpec((1, 1, D), lambda e: (e, 0, 0)),
        ],
        out_specs=pl.BlockSpec((T, D), lambda e: (0, 0)),
        out_shape=jax.ShapeDtypeStruct((T, D), jnp.float32),
        compiler_params=pltpu.CompilerParams(
            dimension_semantics=("arbitrary",)),
    )(x_f, gate_w, w1r, b1g, b1x, dense_2_w, b2)
    return out.reshape(B, L, D)


# speedup vs baseline: 6.5570x; 1.0028x over previous
import functools

import jax
import jax.numpy as jnp
from jax.experimental import pallas as pl
from jax.experimental.pallas import tpu as pltpu

_N_EXPERTS = 8
_FFW = 1024
_FH = _FFW // 2
_D = 768
_LIMIT = 7.0
_ALPHA = 1.702


def _swiglu(hg, hx):
    g = jnp.minimum(hg, _LIMIT)
    xl = jnp.clip(hx, -_LIMIT, _LIMIT)
    return g * jax.nn.sigmoid(_ALPHA * g) * (xl + 1.0)


def _moe_body(x_ref, gw_ref, w1a_ref, w1b_ref, b1g_ref, b1x_ref, w2a_ref,
              w2b_ref, b2_ref, out_ref):
    e = pl.program_id(0)
    x = x_ref[...]

    gate = jnp.dot(x, gw_ref[...], preferred_element_type=jnp.float32)
    cols = jax.lax.broadcasted_iota(jnp.int32, gate.shape, 1)
    m1 = jnp.max(gate, axis=1, keepdims=True)
    a1 = jnp.min(jnp.where(gate == m1, cols, _N_EXPERTS), axis=1, keepdims=True)
    gate2 = jnp.where(cols == a1, -jnp.inf, gate)
    m2 = jnp.max(gate2, axis=1, keepdims=True)
    a2 = jnp.min(jnp.where(gate2 == m2, cols, _N_EXPERTS), axis=1, keepdims=True)
    e2 = jnp.exp(m2 - m1)
    denom = 1.0 + e2
    w_sel = jnp.where(a1 == e, 1.0 / denom,
                      jnp.where(a2 == e, e2 / denom, 0.0))

    dn = (((1,), (1,)), ((), ()))
    h2 = b2_ref[0]
    for w1_ref, w2_ref, lo in ((w1a_ref, w2a_ref, 0), (w1b_ref, w2b_ref, _FH)):
        hg = jax.lax.dot_general(x, w1_ref[0, :, 0:_D], dn,
                                 preferred_element_type=jnp.float32)
        hg = hg + b1g_ref[0, :, lo:lo + _FH]
        hx = jax.lax.dot_general(x, w1_ref[0, :, _D:2 * _D], dn,
                                 preferred_element_type=jnp.float32)
        hx = hx + b1x_ref[0, :, lo:lo + _FH]
        act = _swiglu(hg, hx)
        h2 = h2 + jax.lax.dot_general(act, w2_ref[0], dn,
                                      preferred_element_type=jnp.float32)
    contrib = w_sel * h2

    @pl.when(e == 0)
    def _():
        out_ref[...] = contrib

    @pl.when(e != 0)
    def _():
        out_ref[...] += contrib


@jax.jit
def kernel(x, gate_w, dense_1_w, dense_1_b, dense_2_w, dense_2_b):
    B, L, D = x.shape
    T = B * L
    x_f = x.reshape(T, D)
    w1r = dense_1_w.reshape(_N_EXPERTS, _FFW, 2 * D)
    b1g = dense_1_b[:, 0::2].reshape(_N_EXPERTS, 1, _FFW)
    b1x = dense_1_b[:, 1::2].reshape(_N_EXPERTS, 1, _FFW)
    b2 = dense_2_b.reshape(_N_EXPERTS, 1, _D)

    out = pl.pallas_call(
        _moe_body,
        grid=(_N_EXPERTS,),
        in_specs=[
            pl.BlockSpec((T, D), lambda e: (0, 0)),
            pl.BlockSpec((D, _N_EXPERTS), lambda e: (0, 0)),
            pl.BlockSpec((1, _FH, 2 * D), lambda e: (e, 0, 0)),
            pl.BlockSpec((1, _FH, 2 * D), lambda e: (e, 1, 0)),
            pl.BlockSpec((1, 1, _FFW), lambda e: (e, 0, 0)),
            pl.BlockSpec((1, 1, _FFW), lambda e: (e, 0, 0)),
            pl.BlockSpec((1, D, _FH), lambda e: (e, 0, 0)),
            pl.BlockSpec((1, D, _FH), lambda e: (e, 0, 1)),
            pl.BlockSpec((1, 1, D), lambda e: (e, 0, 0)),
        ],
        out_specs=pl.BlockSpec((T, D), lambda e: (0, 0)),
        out_shape=jax.ShapeDtypeStruct((T, D), jnp.float32),
        compiler_params=pltpu.CompilerParams(
            dimension_semantics=("arbitrary",)),
    )(x_f, gate_w, w1r, w1r, b1g, b1x, dense_2_w, dense_2_w, b2)
    return out.reshape(B, L, D)
